# baseline (speedup 1.0000x reference)
import jax
import jax.numpy as jnp
from jax import lax
from jax.experimental import pallas as pl
from jax.experimental.pallas import tpu as pltpu
from jax.experimental.pallas import tpu_sc as plsc

BASE_VOCAB = 1000000
EXT_VOCAB = 1000
EMBED_DIM = 64
THRESHOLD = 1000000

NUM_CORES = 2
NUM_SUBCORES = 16
LANES = 16
NW = NUM_CORES * NUM_SUBCORES

NBUF = 4
FIRE = NBUF - 2


def _body(tok_hbm, base_hbm, ext_hbm, out_hbm,
          tok_v, bidx_v, rows_v, patch_v, gsem, wsem, psem):
    n_rows, seq = tok_hbm.shape
    rows_per_w = n_rows // NW
    n_win = (seq + LANES - 1) // LANES

    wid = lax.axis_index("s") * NUM_CORES + lax.axis_index("c")
    row0 = wid * rows_per_w

    pltpu.sync_copy(tok_hbm.at[pl.ds(row0, rows_per_w)], tok_v)

    lanes = lax.broadcasted_iota(jnp.int32, (LANES,), 0)

    def win_off(i):
        return jnp.minimum(i * LANES, seq - LANES)

    def compute_bidx(c, b):
        tmax = jnp.zeros((LANES,), jnp.int32)
        for i in range(n_win):
            off = win_off(i)
            t = tok_v[c, pl.ds(off, LANES)]
            tmax = jnp.maximum(tmax, t)
            spread = (row0 + c) * seq + off + lanes
            bidx_v[b, pl.ds(off, LANES)] = jnp.where(t >= THRESHOLD, spread, t)
        return jnp.max(tmax)

    def fire_gather(b):
        pltpu.async_copy(base_hbm.at[bidx_v.at[b]], rows_v.at[b], gsem.at[b])

    def wait_gather(b):
        pltpu.make_async_copy(base_hbm.at[bidx_v.at[b]], rows_v.at[b],
                              gsem.at[b]).wait()

    def fire_write(c, b):
        pltpu.async_copy(rows_v.at[b], out_hbm.at[row0 + c], wsem.at[b])

    def wait_write(b):
        pltpu.make_async_copy(rows_v.at[b], out_hbm.at[0], wsem.at[b]).wait()

    def patch_chunk(c, b):
        @pl.loop(0, n_win)
        def _win(i):
            off = win_off(i)
            t = tok_v[c, pl.ds(off, LANES)]
            m = t >= THRESHOLD

            @pl.when(jnp.max(t) >= THRESHOLD)
            def _patch():
                eidx = jnp.where(m, t - THRESHOLD, 0)
                pltpu.async_copy(ext_hbm.at[eidx], patch_v, psem).wait()
                pos16 = off + lanes

                @pl.loop(0, EMBED_DIM)
                def _col(col):
                    col16 = jnp.full((LANES,), col, jnp.int32)
                    vals = plsc.load_gather(patch_v, [lanes, col16], mask=m)
                    plsc.store_scatter(rows_v.at[b], [pos16, col16], vals, mask=m)

    for b in range(FIRE):
        compute_bidx(b, b)
        fire_gather(b)

    @pl.loop(0, rows_per_w // NBUF)
    def _group(g):
        for b in range(NBUF):
            c = g * NBUF + b
            c_f = c + FIRE
            b_f = (b + FIRE) % NBUF

            @pl.when(c_f < rows_per_w)
            def _fire():
                @pl.when(c_f >= NBUF)
                def _reuse():
                    wait_write(b_f)
                compute_bidx(c_f, b_f)
                fire_gather(b_f)

            wait_gather(b)

            tmax = jnp.int32(0)
            for i in range(n_win):
                t = tok_v[c, pl.ds(win_off(i), LANES)]
                tmax = jnp.maximum(tmax, jnp.max(t))

            @pl.when(tmax >= THRESHOLD)
            def _has_ext():
                patch_chunk(c, b)

            fire_write(c, b)

    for b in range(NBUF):
        wait_write(b)


@jax.jit
def _run(input_tokens, base_table, ext_table):
    mesh = plsc.VectorSubcoreMesh(
        core_axis_name="c", subcore_axis_name="s",
        num_cores=NUM_CORES, num_subcores=NUM_SUBCORES)
    n_rows, seq = input_tokens.shape
    rows_per_w = n_rows // NW
    f = pl.kernel(
        _body,
        out_type=jax.ShapeDtypeStruct((n_rows, seq, EMBED_DIM), jnp.float32),
        mesh=mesh,
        scratch_types=[
            pltpu.VMEM((rows_per_w, seq), jnp.int32),
            pltpu.VMEM((NBUF, seq), jnp.int32),
            pltpu.VMEM((NBUF, seq, EMBED_DIM), jnp.float32),
            pltpu.VMEM((LANES, EMBED_DIM), jnp.float32),
            pltpu.SemaphoreType.DMA((NBUF,)),
            pltpu.SemaphoreType.DMA((NBUF,)),
            pltpu.SemaphoreType.DMA,
        ],
        compiler_params=pltpu.CompilerParams(use_tc_tiling_on_sc=False,
                                             needs_layout_passes=False),
    )
    return f(input_tokens, base_table, ext_table)


def kernel(input_tokens, base_table, ext_table):
    return _run(input_tokens, base_table, ext_table)
